# baseline (speedup 1.0000x reference)
import functools

import jax
import jax.numpy as jnp
from jax import lax
from jax.experimental import pallas as pl
from jax.experimental.pallas import tpu as pltpu
from jax.experimental.pallas import tpu_sc as plsc

_B = 512
_L = 20
_D = 128
_K = 20
_FUT = 40
_NBUCKET = 512


def _encoder_body(xcol_ref, wmat_ref, cb_ref, wih_ref, whh_ref, bih_ref,
                  bhh_ref, state_ref):
    xcol = xcol_ref[...]
    story = jnp.maximum(
        lax.dot_general(xcol, wmat_ref[...], (((1,), (0,)), ((), ()))) +
        cb_ref[...], 0.0)
    gi = lax.dot_general(story, wih_ref[...], (((1,), (1,)), ((), ()))) + \
        bih_ref[...]
    gi = gi.reshape(_L, _B, 3 * _D)
    bhh = bhh_ref[...]
    h = jnp.zeros((_B, _D), jnp.float32)
    for t in range(_L):
        git = gi[t]
        gh = lax.dot_general(h, whh_ref[...], (((1,), (1,)), ((), ()))) + bhh
        r = jax.nn.sigmoid(git[:, :_D] + gh[:, :_D])
        z = jax.nn.sigmoid(git[:, _D:2 * _D] + gh[:, _D:2 * _D])
        n = jnp.tanh(git[:, 2 * _D:] + r * gh[:, 2 * _D:])
        h = (1.0 - z) * n + z * h
    state_ref[...] = h


def _sims_body(sn_ref, pn_ref, table_ref, bmax_ref):
    sn = sn_ref[...]
    pn = pn_ref[...]
    lane = lax.broadcasted_iota(jnp.int32, (_B, 32), 1)
    acc = jnp.zeros((_B, 32), jnp.float32)
    for j in range(32):
        d = lax.dot_general(sn, pn[j * 128:(j + 1) * 128, :],
                            (((1,), (1,)), ((), ())),
                            preferred_element_type=jnp.float32)
        table_ref[j, :, :] = d
        m = jnp.max(d, axis=1, keepdims=True)
        acc = jnp.where(lane == j, m, acc)
    bmax_ref[...] = acc.reshape(1, _B, 32)


def _topk_buckets_body(bmax_ref, out_ref):
    vals = bmax_ref[...]
    col = lax.broadcasted_iota(jnp.int32, vals.shape, 1)
    lane = lax.broadcasted_iota(jnp.int32, (_B, 32), 1)
    acc = jnp.zeros((_B, 32), jnp.int32)
    for i in range(_K):
        m = jnp.max(vals, axis=1, keepdims=True)
        idx = jnp.min(jnp.where(vals == m, col, 2**30), axis=1, keepdims=True)
        acc = jnp.where(lane == i, idx, acc)
        vals = jnp.where(col == idx, -3.0e38, vals)
    out_ref[...] = acc


def _final_topk_body(cand_ref, oc_ref, out_ref):
    vals = cand_ref[...]
    oc = oc_ref[...]
    col = lax.broadcasted_iota(jnp.int32, vals.shape, 1)
    lane = lax.broadcasted_iota(jnp.int32, (_B, 32), 1)
    acc = jnp.zeros((_B, 32), jnp.int32)
    for i in range(_K):
        m = jnp.max(vals, axis=1, keepdims=True)
        pos = jnp.min(jnp.where(vals == m, col, 2**30), axis=1, keepdims=True)
        orig = jnp.max(jnp.where(col == pos, oc, -1), axis=1, keepdims=True)
        acc = jnp.where(lane == i, orig, acc)
        vals = jnp.where(col == pos, -3.0e38, vals)
    out_ref[...] = acc


def _decoder_body(info_ref, p0_ref, wih_ref, waug_ref, bih_ref, bhh_ref,
                  fcw_ref, fcb_ref, out_ref):
    it = info_ref[...]
    waug = waug_ref[...]
    bih = bih_ref[...]
    bhh = bhh_ref[...]
    fcb = fcb_ref[...]
    H = 2 * _D
    G = 3 * H
    blk = it.shape[0]
    hf = blk // 2

    def gates(gi_like, gh_like, h_prev):
        r = jax.nn.sigmoid(gi_like[:, :H] + gh_like[:, :H])
        z = jax.nn.sigmoid(gi_like[:, H:2 * H] + gh_like[:, H:2 * H])
        n = jnp.tanh(gi_like[:, 2 * H:] + r * gh_like[:, 2 * H:])
        return (1.0 - z) * n + z * h_prev

    gi = lax.dot_general(it, wih_ref[...], (((1,), (1,)), ((), ()))) + bih
    z0 = jnp.zeros((blk, H), jnp.float32)
    h = gates(gi, bhh, z0)
    hA, hB = h[:hf], h[hf:]
    presA = p0_ref[:hf, :]
    presB = p0_ref[hf:, :]
    for t in range(1, _FUT):
        augA = lax.dot_general(hA, waug, (((1,), (1,)), ((), ())))
        augB = lax.dot_general(hB, waug, (((1,), (1,)), ((), ())))
        presA = presA + (augA[:, G:G + 2] + fcb)
        presB = presB + (augB[:, G:G + 2] + fcb)
        out_ref[:hf, 2 * (t - 1):2 * t] = presA
        out_ref[hf:, 2 * (t - 1):2 * t] = presB
        hA = gates(bih, augA[:, :G] + bhh, hA)
        hB = gates(bih, augB[:, :G] + bhh, hB)
    dA = lax.dot_general(hA, fcw_ref[...], (((1,), (1,)), ((), ())))
    dB = lax.dot_general(hB, fcw_ref[...], (((1,), (1,)), ((), ())))
    presA = presA + (dA + fcb)
    presB = presB + (dB + fcb)
    out_ref[:hf, 2 * _FUT - 2:2 * _FUT] = presA
    out_ref[hf:, 2 * _FUT - 2:2 * _FUT] = presB


def _sc_gather(table, idx3):
    n_rows = idx3.shape[0] * idx3.shape[1] * idx3.shape[2]
    d = table.shape[1]
    info = plsc.get_sparse_core_info()
    nc, ns = info.num_cores, info.num_subcores
    cpw, ch = idx3.shape[1], idx3.shape[2]
    mesh = plsc.VectorSubcoreMesh(core_axis_name="c", subcore_axis_name="s")

    @functools.partial(
        pl.kernel, mesh=mesh,
        out_type=jax.ShapeDtypeStruct((n_rows, d), jnp.float32),
        scratch_types=[pltpu.VMEM((cpw, ch), jnp.int32),
                       pltpu.VMEM((cpw * ch, d), jnp.float32),
                       pltpu.SemaphoreType.DMA])
    def k(table_hbm, idx_hbm, out_hbm, idx_v, rows_v, sem):
        wid = lax.axis_index("s") * nc + lax.axis_index("c")
        pltpu.sync_copy(idx_hbm.at[wid], idx_v)
        cps = [pltpu.async_copy(table_hbm.at[idx_v.at[j]],
                                rows_v.at[pl.ds(j * ch, ch)], sem)
               for j in range(cpw)]
        for c in cps:
            c.wait()
        pltpu.sync_copy(rows_v, out_hbm.at[pl.ds(wid * cpw * ch, cpw * ch)])

    return k(table, idx3)


def kernel(past, conv_w, conv_b, enc_Wih, enc_Whh, enc_bih, enc_bhh,
           memory_past, memory_fut, dec_Wih, dec_Whh, dec_bih, dec_bhh,
           fc_w, fc_b):
    f32 = jnp.float32
    M = memory_past.shape[0]

    p = jnp.pad(past, ((0, 0), (1, 1), (0, 0)))
    xcol = jnp.stack([p[:, t:t + 3, :].reshape(_B, 6) for t in range(_L)],
                     axis=0).reshape(_L * _B, 6)
    xcol = jnp.pad(xcol, ((0, 0), (0, 2)))
    wmat = jnp.pad(jnp.transpose(conv_w, (2, 1, 0)).reshape(6, _D),
                   ((0, 2), (0, 0)))

    state = pl.pallas_call(
        _encoder_body,
        out_shape=jax.ShapeDtypeStruct((_B, _D), f32),
    )(xcol, wmat, conv_b.reshape(1, _D), enc_Wih, enc_Whh,
      enc_bih.reshape(1, 3 * _D), enc_bhh.reshape(1, 3 * _D))

    snorm = state / (jnp.linalg.norm(state, axis=1, keepdims=True) + 1e-12)
    pnorm = memory_past / (
        jnp.linalg.norm(memory_past, axis=1, keepdims=True) + 1e-12)

    nblk = 16
    mb = M // nblk
    table, bmax3 = pl.pallas_call(
        _sims_body,
        grid=(nblk,),
        in_specs=[pl.BlockSpec((_B, _D), lambda i: (0, 0)),
                  pl.BlockSpec((mb, _D), lambda i: (i, 0))],
        out_specs=[pl.BlockSpec((32, _B, _D), lambda i: (i, 0, 0)),
                   pl.BlockSpec((1, _B, 32), lambda i: (i, 0, 0))],
        out_shape=(jax.ShapeDtypeStruct((_NBUCKET, _B, _D), f32),
                   jax.ShapeDtypeStruct((nblk, _B, 32), f32)),
    )(snorm, pnorm)
    bmax = jnp.transpose(bmax3, (1, 0, 2)).reshape(_B, _NBUCKET)

    buckets = pl.pallas_call(
        _topk_buckets_body,
        out_shape=jax.ShapeDtypeStruct((_B, 32), jnp.int32))(bmax)
    b20 = buckets[:, :_K]

    gidx = (b20 * _B +
            jnp.arange(_B, dtype=jnp.int32)[:, None]).reshape(32, -1, 80)
    cand = _sc_gather(table.reshape(_B * _NBUCKET, _D), gidx)
    cand2 = cand.reshape(_B, _K * _D)
    ocol = (jnp.repeat(b20, _D, axis=1) * _D +
            jnp.tile(jnp.arange(_D, dtype=jnp.int32), _K)[None, :])

    midx = pl.pallas_call(
        _final_topk_body,
        out_shape=jax.ShapeDtypeStruct((_B, 32), jnp.int32))(cand2, ocol)
    ind = midx[:, :_K].reshape(32, -1, 80)

    fut = _sc_gather(memory_fut, ind)

    info = jnp.concatenate([jnp.repeat(state, _K, axis=0), fut], axis=1)
    p0 = jnp.repeat(past[:, -1, :], _K, axis=0)
    waug = jnp.concatenate([dec_Whh, fc_w], axis=0)
    blk = 2048
    g = (_B * _K) // blk
    out = pl.pallas_call(
        _decoder_body,
        grid=(g,),
        in_specs=[pl.BlockSpec((blk, 2 * _D), lambda i: (i, 0)),
                  pl.BlockSpec((blk, 2), lambda i: (i, 0)),
                  pl.BlockSpec((6 * _D, 2 * _D), lambda i: (0, 0)),
                  pl.BlockSpec((6 * _D + 2, 2 * _D), lambda i: (0, 0)),
                  pl.BlockSpec((1, 6 * _D), lambda i: (0, 0)),
                  pl.BlockSpec((1, 6 * _D), lambda i: (0, 0)),
                  pl.BlockSpec((2, 2 * _D), lambda i: (0, 0)),
                  pl.BlockSpec((1, 2), lambda i: (0, 0))],
        out_specs=pl.BlockSpec((blk, 2 * _FUT), lambda i: (i, 0)),
        out_shape=jax.ShapeDtypeStruct((_B * _K, 2 * _FUT), f32),
    )(info, p0, dec_Wih, waug, dec_bih.reshape(1, 6 * _D),
      dec_bhh.reshape(1, 6 * _D), fc_w, fc_b.reshape(1, 2))

    return out.reshape(_B, _K, _FUT, 2)


# speedup vs baseline: 1.0070x; 1.0070x over previous
import functools

import jax
import jax.numpy as jnp
from jax import lax
from jax.experimental import pallas as pl
from jax.experimental.pallas import tpu as pltpu
from jax.experimental.pallas import tpu_sc as plsc

_B = 512
_L = 20
_D = 128
_K = 20
_FUT = 40
_NBUCKET = 512


def _encoder_body(xcol_ref, wmat_ref, cb_ref, wih_ref, whh_ref, bih_ref,
                  bhh_ref, state_ref):
    xcol = xcol_ref[...]
    story = jnp.maximum(
        lax.dot_general(xcol, wmat_ref[...], (((1,), (0,)), ((), ()))) +
        cb_ref[...], 0.0)
    gi = lax.dot_general(story, wih_ref[...], (((1,), (1,)), ((), ()))) + \
        bih_ref[...]
    gi = gi.reshape(_L, _B, 3 * _D)
    bhh = bhh_ref[...]
    h = jnp.zeros((_B, _D), jnp.float32)
    for t in range(_L):
        git = gi[t]
        gh = lax.dot_general(h, whh_ref[...], (((1,), (1,)), ((), ()))) + bhh
        r = jax.nn.sigmoid(git[:, :_D] + gh[:, :_D])
        z = jax.nn.sigmoid(git[:, _D:2 * _D] + gh[:, _D:2 * _D])
        n = jnp.tanh(git[:, 2 * _D:] + r * gh[:, 2 * _D:])
        h = (1.0 - z) * n + z * h
    state_ref[...] = h


def _sims_body(sn_ref, pn_ref, table_ref, bmax_ref):
    sn = sn_ref[...]
    pn = pn_ref[...]
    lane = lax.broadcasted_iota(jnp.int32, (_B, 32), 1)
    acc = jnp.zeros((_B, 32), jnp.float32)
    for j in range(32):
        d = lax.dot_general(sn, pn[j * 128:(j + 1) * 128, :],
                            (((1,), (1,)), ((), ())),
                            preferred_element_type=jnp.float32)
        table_ref[j, :, :] = d
        m = jnp.max(d, axis=1, keepdims=True)
        acc = jnp.where(lane == j, m, acc)
    bmax_ref[...] = acc.reshape(1, _B, 32)


def _topk_buckets_body(bmax_ref, out_ref):
    vals = bmax_ref[...]
    col = lax.broadcasted_iota(jnp.int32, vals.shape, 1)
    lane = lax.broadcasted_iota(jnp.int32, (_B, 32), 1)
    acc = jnp.zeros((_B, 32), jnp.int32)
    for i in range(_K):
        m = jnp.max(vals, axis=1, keepdims=True)
        idx = jnp.min(jnp.where(vals == m, col, 2**30), axis=1, keepdims=True)
        acc = jnp.where(lane == i, idx, acc)
        vals = jnp.where(col == idx, -3.0e38, vals)
    out_ref[...] = acc


def _final_topk_body(cand_ref, oc_ref, out_ref):
    vals = cand_ref[...]
    oc = oc_ref[...]
    lane = lax.broadcasted_iota(jnp.int32, (_B, 32), 1)
    acc = jnp.zeros((_B, 32), jnp.int32)
    for i in range(_K):
        m = jnp.max(vals, axis=1, keepdims=True)
        orig = jnp.min(jnp.where(vals == m, oc, 2**30), axis=1, keepdims=True)
        acc = jnp.where(lane == i, orig, acc)
        vals = jnp.where(oc == orig, -3.0e38, vals)
    out_ref[...] = acc


def _decoder_body(info_ref, p0_ref, wih_ref, waug_ref, bih_ref, bhh_ref,
                  fcw_ref, fcb_ref, out_ref):
    it = info_ref[...]
    waug = waug_ref[...]
    bih = bih_ref[...]
    bhh = bhh_ref[...]
    fcb = fcb_ref[...]
    H = 2 * _D
    G = 3 * H
    gi = lax.dot_general(it, wih_ref[...], (((1,), (1,)), ((), ()))) + bih
    r = jax.nn.sigmoid(gi[:, :H] + bhh[:, :H])
    z = jax.nn.sigmoid(gi[:, H:2 * H] + bhh[:, H:2 * H])
    n = jnp.tanh(gi[:, 2 * H:] + r * bhh[:, 2 * H:])
    h = (1.0 - z) * n
    pres = p0_ref[...]
    for t in range(1, _FUT):
        aug = lax.dot_general(h, waug, (((1,), (1,)), ((), ())))
        pres = pres + (aug[:, G:G + 2] + fcb)
        out_ref[:, 2 * (t - 1):2 * t] = pres
        gh = aug[:, :G] + bhh
        r = jax.nn.sigmoid(bih[:, :H] + gh[:, :H])
        z = jax.nn.sigmoid(bih[:, H:2 * H] + gh[:, H:2 * H])
        n = jnp.tanh(bih[:, 2 * H:] + r * gh[:, 2 * H:])
        h = (1.0 - z) * n + z * h
    d_last = lax.dot_general(h, fcw_ref[...], (((1,), (1,)), ((), ())))
    pres = pres + (d_last + fcb)
    out_ref[:, 2 * _FUT - 2:2 * _FUT] = pres


def _sc_gather(table, idx3):
    n_rows = idx3.shape[0] * idx3.shape[1] * idx3.shape[2]
    d = table.shape[1]
    info = plsc.get_sparse_core_info()
    nc, ns = info.num_cores, info.num_subcores
    cpw, ch = idx3.shape[1], idx3.shape[2]
    mesh = plsc.VectorSubcoreMesh(core_axis_name="c", subcore_axis_name="s")

    @functools.partial(
        pl.kernel, mesh=mesh,
        out_type=jax.ShapeDtypeStruct((n_rows, d), jnp.float32),
        scratch_types=[pltpu.VMEM((cpw, ch), jnp.int32),
                       pltpu.VMEM((cpw * ch, d), jnp.float32),
                       pltpu.SemaphoreType.DMA])
    def k(table_hbm, idx_hbm, out_hbm, idx_v, rows_v, sem):
        wid = lax.axis_index("s") * nc + lax.axis_index("c")
        pltpu.sync_copy(idx_hbm.at[wid], idx_v)
        cps = [pltpu.async_copy(table_hbm.at[idx_v.at[j]],
                                rows_v.at[pl.ds(j * ch, ch)], sem)
               for j in range(cpw)]
        for c in cps:
            c.wait()
        pltpu.sync_copy(rows_v, out_hbm.at[pl.ds(wid * cpw * ch, cpw * ch)])

    return k(table, idx3)


def kernel(past, conv_w, conv_b, enc_Wih, enc_Whh, enc_bih, enc_bhh,
           memory_past, memory_fut, dec_Wih, dec_Whh, dec_bih, dec_bhh,
           fc_w, fc_b):
    f32 = jnp.float32
    M = memory_past.shape[0]

    p = jnp.pad(past, ((0, 0), (1, 1), (0, 0)))
    xcol = jnp.stack([p[:, t:t + 3, :].reshape(_B, 6) for t in range(_L)],
                     axis=0).reshape(_L * _B, 6)
    xcol = jnp.pad(xcol, ((0, 0), (0, 2)))
    wmat = jnp.pad(jnp.transpose(conv_w, (2, 1, 0)).reshape(6, _D),
                   ((0, 2), (0, 0)))

    state = pl.pallas_call(
        _encoder_body,
        out_shape=jax.ShapeDtypeStruct((_B, _D), f32),
    )(xcol, wmat, conv_b.reshape(1, _D), enc_Wih, enc_Whh,
      enc_bih.reshape(1, 3 * _D), enc_bhh.reshape(1, 3 * _D))

    snorm = state / (jnp.linalg.norm(state, axis=1, keepdims=True) + 1e-12)
    pnorm = memory_past / (
        jnp.linalg.norm(memory_past, axis=1, keepdims=True) + 1e-12)

    nblk = 16
    mb = M // nblk
    table, bmax3 = pl.pallas_call(
        _sims_body,
        grid=(nblk,),
        in_specs=[pl.BlockSpec((_B, _D), lambda i: (0, 0)),
                  pl.BlockSpec((mb, _D), lambda i: (i, 0))],
        out_specs=[pl.BlockSpec((32, _B, _D), lambda i: (i, 0, 0)),
                   pl.BlockSpec((1, _B, 32), lambda i: (i, 0, 0))],
        out_shape=(jax.ShapeDtypeStruct((_NBUCKET, _B, _D), f32),
                   jax.ShapeDtypeStruct((nblk, _B, 32), f32)),
    )(snorm, pnorm)
    bmax = jnp.transpose(bmax3, (1, 0, 2)).reshape(_B, _NBUCKET)

    buckets = pl.pallas_call(
        _topk_buckets_body,
        out_shape=jax.ShapeDtypeStruct((_B, 32), jnp.int32))(bmax)
    b20 = buckets[:, :_K]

    gidx = (b20 * _B +
            jnp.arange(_B, dtype=jnp.int32)[:, None]).reshape(32, -1, 80)
    cand = _sc_gather(table.reshape(_B * _NBUCKET, _D), gidx)
    cand2 = cand.reshape(_B, _K * _D)
    ocol = (jnp.repeat(b20, _D, axis=1) * _D +
            jnp.tile(jnp.arange(_D, dtype=jnp.int32), _K)[None, :])

    midx = pl.pallas_call(
        _final_topk_body,
        out_shape=jax.ShapeDtypeStruct((_B, 32), jnp.int32))(cand2, ocol)
    ind = midx[:, :_K].reshape(32, -1, 80)

    fut = _sc_gather(memory_fut, ind)

    info = jnp.concatenate([jnp.repeat(state, _K, axis=0), fut], axis=1)
    p0 = jnp.repeat(past[:, -1, :], _K, axis=0)
    waug = jnp.concatenate([dec_Whh, fc_w], axis=0)
    blk = 2048
    g = (_B * _K) // blk
    out = pl.pallas_call(
        _decoder_body,
        grid=(g,),
        in_specs=[pl.BlockSpec((blk, 2 * _D), lambda i: (i, 0)),
                  pl.BlockSpec((blk, 2), lambda i: (i, 0)),
                  pl.BlockSpec((6 * _D, 2 * _D), lambda i: (0, 0)),
                  pl.BlockSpec((6 * _D + 2, 2 * _D), lambda i: (0, 0)),
                  pl.BlockSpec((1, 6 * _D), lambda i: (0, 0)),
                  pl.BlockSpec((1, 6 * _D), lambda i: (0, 0)),
                  pl.BlockSpec((2, 2 * _D), lambda i: (0, 0)),
                  pl.BlockSpec((1, 2), lambda i: (0, 0))],
        out_specs=pl.BlockSpec((blk, 2 * _FUT), lambda i: (i, 0)),
        out_shape=jax.ShapeDtypeStruct((_B * _K, 2 * _FUT), f32),
    )(info, p0, dec_Wih, waug, dec_bih.reshape(1, 6 * _D),
      dec_bhh.reshape(1, 6 * _D), fc_w, fc_b.reshape(1, 2))

    return out.reshape(_B, _K, _FUT, 2)


# speedup vs baseline: 1.0142x; 1.0072x over previous
import functools

import jax
import jax.numpy as jnp
from jax import lax
from jax.experimental import pallas as pl
from jax.experimental.pallas import tpu as pltpu
from jax.experimental.pallas import tpu_sc as plsc

_B = 512
_L = 20
_D = 128
_K = 20
_FUT = 40
_NBUCKET = 512


def _encoder_body(xcol_ref, wmat_ref, cb_ref, wih_ref, whh_ref, bih_ref,
                  bhh_ref, state_ref):
    xcol = xcol_ref[...]
    story = jnp.maximum(
        lax.dot_general(xcol, wmat_ref[...], (((1,), (0,)), ((), ()))) +
        cb_ref[...], 0.0)
    gi = lax.dot_general(story, wih_ref[...], (((1,), (1,)), ((), ()))) + \
        bih_ref[...]
    gi = gi.reshape(_L, _B, 3 * _D)
    bhh = bhh_ref[...]
    h = jnp.zeros((_B, _D), jnp.float32)
    for t in range(_L):
        git = gi[t]
        gh = lax.dot_general(h, whh_ref[...], (((1,), (1,)), ((), ()))) + bhh
        r = jax.nn.sigmoid(git[:, :_D] + gh[:, :_D])
        z = jax.nn.sigmoid(git[:, _D:2 * _D] + gh[:, _D:2 * _D])
        n = jnp.tanh(git[:, 2 * _D:] + r * gh[:, 2 * _D:])
        h = (1.0 - z) * n + z * h
    state_ref[...] = h


def _sims_body(sn_ref, pn_ref, table_ref, bmax_ref):
    sn = sn_ref[...]
    pn = pn_ref[...]
    lane = lax.broadcasted_iota(jnp.int32, (_B, 32), 1)
    acc = jnp.zeros((_B, 32), jnp.float32)
    for j in range(32):
        d = lax.dot_general(sn, pn[j * 128:(j + 1) * 128, :],
                            (((1,), (1,)), ((), ())),
                            preferred_element_type=jnp.float32)
        table_ref[j, :, :] = d
        m = jnp.max(d, axis=1, keepdims=True)
        acc = jnp.where(lane == j, m, acc)
    bmax_ref[...] = acc.reshape(1, _B, 32)


def _topk_buckets_body(bmax_ref, out_ref):
    vals = bmax_ref[...]
    col = lax.broadcasted_iota(jnp.int32, vals.shape, 1)
    lane = lax.broadcasted_iota(jnp.int32, (_B, 32), 1)
    acc = jnp.zeros((_B, 32), jnp.int32)
    for i in range(_K):
        m = jnp.max(vals, axis=1, keepdims=True)
        idx = jnp.min(jnp.where(vals == m, col, 2**30), axis=1, keepdims=True)
        acc = jnp.where(lane == i, idx, acc)
        vals = jnp.where(col == idx, -3.0e38, vals)
    out_ref[...] = acc


def _final_topk_body(cand_ref, oc_ref, out_ref):
    vals = cand_ref[...]
    oc = oc_ref[...]
    col = lax.broadcasted_iota(jnp.int32, vals.shape, 1)
    lane = lax.broadcasted_iota(jnp.int32, (_B, 32), 1)
    acc = jnp.zeros((_B, 32), jnp.int32)
    for i in range(_K):
        m = jnp.max(vals, axis=1, keepdims=True)
        pos = jnp.min(jnp.where(vals == m, col, 2**30), axis=1, keepdims=True)
        orig = jnp.max(jnp.where(col == pos, oc, -1), axis=1, keepdims=True)
        acc = jnp.where(lane == i, orig, acc)
        vals = jnp.where(col == pos, -3.0e38, vals)
    out_ref[...] = acc


def _decoder_body(info_ref, p0_ref, wih_ref, waug_ref, bih_ref, bhh_ref,
                  fcw_ref, fcb_ref, out_ref):
    it = info_ref[...]
    waug = waug_ref[...]
    bih = bih_ref[...]
    bhh = bhh_ref[...]
    fcb = fcb_ref[...]
    H = 2 * _D
    G = 3 * H
    gi = lax.dot_general(it, wih_ref[...], (((1,), (1,)), ((), ()))) + bih
    r = jax.nn.sigmoid(gi[:, :H] + bhh[:, :H])
    z = jax.nn.sigmoid(gi[:, H:2 * H] + bhh[:, H:2 * H])
    n = jnp.tanh(gi[:, 2 * H:] + r * bhh[:, 2 * H:])
    h = (1.0 - z) * n
    pres = p0_ref[...]
    for t in range(1, _FUT):
        aug = lax.dot_general(h, waug, (((1,), (1,)), ((), ())))
        pres = pres + (aug[:, G:G + 2] + fcb)
        out_ref[:, 2 * (t - 1):2 * t] = pres
        gh = aug[:, :G] + bhh
        r = jax.nn.sigmoid(bih[:, :H] + gh[:, :H])
        z = jax.nn.sigmoid(bih[:, H:2 * H] + gh[:, H:2 * H])
        n = jnp.tanh(bih[:, 2 * H:] + r * gh[:, 2 * H:])
        h = (1.0 - z) * n + z * h
    d_last = lax.dot_general(h, fcw_ref[...], (((1,), (1,)), ((), ())))
    pres = pres + (d_last + fcb)
    out_ref[:, 2 * _FUT - 2:2 * _FUT] = pres


def _sc_gather(table, idx3):
    n_rows = idx3.shape[0] * idx3.shape[1] * idx3.shape[2]
    d = table.shape[1]
    info = plsc.get_sparse_core_info()
    nc, ns = info.num_cores, info.num_subcores
    cpw, ch = idx3.shape[1], idx3.shape[2]
    mesh = plsc.VectorSubcoreMesh(core_axis_name="c", subcore_axis_name="s")

    @functools.partial(
        pl.kernel, mesh=mesh,
        out_type=jax.ShapeDtypeStruct((n_rows, d), jnp.float32),
        scratch_types=[pltpu.VMEM((cpw, ch), jnp.int32),
                       pltpu.VMEM((cpw * ch, d), jnp.float32),
                       pltpu.SemaphoreType.DMA])
    def k(table_hbm, idx_hbm, out_hbm, idx_v, rows_v, sem):
        wid = lax.axis_index("s") * nc + lax.axis_index("c")
        pltpu.sync_copy(idx_hbm.at[wid], idx_v)
        cps = [pltpu.async_copy(table_hbm.at[idx_v.at[j]],
                                rows_v.at[pl.ds(j * ch, ch)], sem)
               for j in range(cpw)]
        for c in cps:
            c.wait()
        pltpu.sync_copy(rows_v, out_hbm.at[pl.ds(wid * cpw * ch, cpw * ch)])

    return k(table, idx3)


def kernel(past, conv_w, conv_b, enc_Wih, enc_Whh, enc_bih, enc_bhh,
           memory_past, memory_fut, dec_Wih, dec_Whh, dec_bih, dec_bhh,
           fc_w, fc_b):
    f32 = jnp.float32
    M = memory_past.shape[0]

    p = jnp.pad(past, ((0, 0), (1, 1), (0, 0)))
    xcol = jnp.stack([p[:, t:t + 3, :].reshape(_B, 6) for t in range(_L)],
                     axis=0).reshape(_L * _B, 6)
    xcol = jnp.pad(xcol, ((0, 0), (0, 2)))
    wmat = jnp.pad(jnp.transpose(conv_w, (2, 1, 0)).reshape(6, _D),
                   ((0, 2), (0, 0)))

    state = pl.pallas_call(
        _encoder_body,
        out_shape=jax.ShapeDtypeStruct((_B, _D), f32),
    )(xcol, wmat, conv_b.reshape(1, _D), enc_Wih, enc_Whh,
      enc_bih.reshape(1, 3 * _D), enc_bhh.reshape(1, 3 * _D))

    snorm = state / (jnp.linalg.norm(state, axis=1, keepdims=True) + 1e-12)
    pnorm = memory_past / (
        jnp.linalg.norm(memory_past, axis=1, keepdims=True) + 1e-12)

    nblk = 16
    mb = M // nblk
    table, bmax3 = pl.pallas_call(
        _sims_body,
        grid=(nblk,),
        in_specs=[pl.BlockSpec((_B, _D), lambda i: (0, 0)),
                  pl.BlockSpec((mb, _D), lambda i: (i, 0))],
        out_specs=[pl.BlockSpec((32, _B, _D), lambda i: (i, 0, 0)),
                   pl.BlockSpec((1, _B, 32), lambda i: (i, 0, 0))],
        out_shape=(jax.ShapeDtypeStruct((_NBUCKET, _B, _D), f32),
                   jax.ShapeDtypeStruct((nblk, _B, 32), f32)),
    )(snorm, pnorm)
    bmax = jnp.transpose(bmax3, (1, 0, 2)).reshape(_B, _NBUCKET)

    buckets = pl.pallas_call(
        _topk_buckets_body,
        out_shape=jax.ShapeDtypeStruct((_B, 32), jnp.int32))(bmax)
    b20 = buckets[:, :_K]

    gidx = (b20 * _B +
            jnp.arange(_B, dtype=jnp.int32)[:, None]).reshape(32, -1, 80)
    cand = _sc_gather(table.reshape(_B * _NBUCKET, _D), gidx)
    cand2 = cand.reshape(_B, _K * _D)
    ocol = (jnp.repeat(b20, _D, axis=1) * _D +
            jnp.tile(jnp.arange(_D, dtype=jnp.int32), _K)[None, :])

    midx = pl.pallas_call(
        _final_topk_body,
        out_shape=jax.ShapeDtypeStruct((_B, 32), jnp.int32))(cand2, ocol)
    ind = midx[:, :_K].reshape(32, -1, 80)

    fut = _sc_gather(memory_fut, ind)

    info = jnp.concatenate([jnp.repeat(state, _K, axis=0), fut], axis=1)
    p0 = jnp.repeat(past[:, -1, :], _K, axis=0)
    waug = jnp.concatenate([dec_Whh, fc_w], axis=0)
    blk = 2048
    g = (_B * _K) // blk
    out = pl.pallas_call(
        _decoder_body,
        grid=(g,),
        in_specs=[pl.BlockSpec((blk, 2 * _D), lambda i: (i, 0)),
                  pl.BlockSpec((blk, 2), lambda i: (i, 0)),
                  pl.BlockSpec((6 * _D, 2 * _D), lambda i: (0, 0)),
                  pl.BlockSpec((6 * _D + 2, 2 * _D), lambda i: (0, 0)),
                  pl.BlockSpec((1, 6 * _D), lambda i: (0, 0)),
                  pl.BlockSpec((1, 6 * _D), lambda i: (0, 0)),
                  pl.BlockSpec((2, 2 * _D), lambda i: (0, 0)),
                  pl.BlockSpec((1, 2), lambda i: (0, 0))],
        out_specs=pl.BlockSpec((blk, 2 * _FUT), lambda i: (i, 0)),
        out_shape=jax.ShapeDtypeStruct((_B * _K, 2 * _FUT), f32),
    )(info, p0, dec_Wih, waug, dec_bih.reshape(1, 6 * _D),
      dec_bhh.reshape(1, 6 * _D), fc_w, fc_b.reshape(1, 2))

    return out.reshape(_B, _K, _FUT, 2)
